# initial kernel scaffold (unmeasured)
import jax
import jax.numpy as jnp
from jax import lax
from jax.experimental import pallas as pl
from jax.experimental.pallas import tpu as pltpu

N_DEV = 32
M_PER = 256
K = 8192
N = 4096
BN = 512


def _a2a_body(x_ref, out_ref, send_sems, recv_sems):
    me = lax.axis_index("i")

    out_ref[me] = x_ref[pl.ds(me * M_PER, M_PER), :]

    rdmas = []
    for o in range(1, N_DEV):
        peer = lax.rem(me + o, N_DEV)
        rdma = pltpu.make_async_remote_copy(
            src_ref=x_ref.at[pl.ds(peer * M_PER, M_PER), :],
            dst_ref=out_ref.at[me],
            send_sem=send_sems.at[o],
            recv_sem=recv_sems.at[o],
            device_id=(peer,),
            device_id_type=pl.DeviceIdType.MESH,
        )
        rdma.start()
        rdmas.append(rdma)
    for rdma in rdmas:
        rdma.wait()


def _a2a(x):
    return pl.pallas_call(
        _a2a_body,
        out_shape=jax.ShapeDtypeStruct((N_DEV, M_PER, M_PER), jnp.float32),
        in_specs=[pl.BlockSpec(memory_space=pltpu.VMEM)],
        out_specs=pl.BlockSpec(memory_space=pltpu.VMEM),
        scratch_shapes=[
            pltpu.SemaphoreType.DMA((N_DEV,)),
            pltpu.SemaphoreType.DMA((N_DEV,)),
        ],
        compiler_params=pltpu.CompilerParams(collective_id=0),
    )(x)


def _gelu(y):
    c = 0.7978845608028654
    return 0.5 * y * (1.0 + jnp.tanh(c * (y + 0.044715 * y * y * y)))


def _gemm_body(a_ref, w_ref, o_ref, acc_ref):
    k = pl.program_id(1)

    @pl.when(k == 0)
    def _():
        acc_ref[...] = jnp.zeros_like(acc_ref)

    acc_ref[...] += jnp.dot(
        a_ref[k], w_ref[...], preferred_element_type=jnp.float32
    )

    @pl.when(k == N_DEV - 1)
    def _():
        o_ref[...] = _gelu(acc_ref[...])


def _gemm(a, w):
    return pl.pallas_call(
        _gemm_body,
        grid=(N // BN, N_DEV),
        in_specs=[
            pl.BlockSpec((N_DEV, M_PER, M_PER), lambda n, k: (0, 0, 0)),
            pl.BlockSpec((M_PER, BN), lambda n, k: (k, n)),
        ],
        out_specs=pl.BlockSpec((M_PER, BN), lambda n, k: (0, n)),
        out_shape=jax.ShapeDtypeStruct((M_PER, N), jnp.float32),
        scratch_shapes=[pltpu.VMEM((M_PER, BN), jnp.float32)],
    )(a, w)


def kernel(x, w_mat):
    return _gemm(_a2a(x), w_mat)


# baseline (device time: 307122 ns/iter reference)
import jax
import jax.numpy as jnp
from jax import lax
from jax.experimental import pallas as pl
from jax.experimental.pallas import tpu as pltpu

N_DEV = 32
M_PER = 256
K = 8192
N = 4096
BN = 512


def _a2a_body(x_ref, out_ref, send_sems, recv_sems):
    me = lax.axis_index("i")

    out_ref[me] = x_ref[pl.ds(me * M_PER, M_PER), :]

    rdmas = []
    for o in range(1, N_DEV):
        peer = lax.rem(me + o, N_DEV)
        rdma = pltpu.make_async_remote_copy(
            src_ref=x_ref.at[pl.ds(peer * M_PER, M_PER), :],
            dst_ref=out_ref.at[me],
            send_sem=send_sems.at[o],
            recv_sem=recv_sems.at[o],
            device_id=(peer,),
            device_id_type=pl.DeviceIdType.MESH,
        )
        rdma.start()
        rdmas.append(rdma)
    for rdma in rdmas:
        rdma.wait()


def _a2a(x):
    return pl.pallas_call(
        _a2a_body,
        out_shape=jax.ShapeDtypeStruct((N_DEV, M_PER, M_PER), jnp.float32),
        in_specs=[pl.BlockSpec(memory_space=pltpu.VMEM)],
        out_specs=pl.BlockSpec(memory_space=pltpu.VMEM),
        scratch_shapes=[
            pltpu.SemaphoreType.DMA((N_DEV,)),
            pltpu.SemaphoreType.DMA((N_DEV,)),
        ],
    )(x)


def _gelu(y):
    c = 0.7978845608028654
    return 0.5 * y * (1.0 + jnp.tanh(c * (y + 0.044715 * y * y * y)))


def _gemm_body(a_ref, w_ref, o_ref, acc_ref):
    k = pl.program_id(1)

    @pl.when(k == 0)
    def _():
        acc_ref[...] = jnp.zeros_like(acc_ref)

    acc_ref[...] += jnp.dot(
        a_ref[k], w_ref[...], preferred_element_type=jnp.float32
    )

    @pl.when(k == N_DEV - 1)
    def _():
        o_ref[...] = _gelu(acc_ref[...])


def _gemm(a, w):
    return pl.pallas_call(
        _gemm_body,
        grid=(N // BN, N_DEV),
        in_specs=[
            pl.BlockSpec((N_DEV, M_PER, M_PER), lambda n, k: (0, 0, 0)),
            pl.BlockSpec((M_PER, BN), lambda n, k: (k, n)),
        ],
        out_specs=pl.BlockSpec((M_PER, BN), lambda n, k: (0, n)),
        out_shape=jax.ShapeDtypeStruct((M_PER, N), jnp.float32),
        scratch_shapes=[pltpu.VMEM((M_PER, BN), jnp.float32)],
    )(a, w)


def kernel(x, w_mat):
    return _gemm(_a2a(x), w_mat)
